# baseline (speedup 1.0000x reference)
import functools
import math

import jax
import jax.numpy as jnp
from jax import lax
from jax.experimental import pallas as pl
from jax.experimental.pallas import tpu as pltpu
from jax.experimental.pallas import tpu_sc as plsc

N_EXP = 8
TOP_K = 2
TRAIN_CAPACITY = 1.25
MIN_CAPACITY = 4

SC_CORES = 1
SC_SUBCORES = 16
SC_LANES = 16
SC_TILES = SC_CORES * SC_SUBCORES


def _capacity(num_tokens: int) -> int:
    cap = math.floor(TOP_K * TRAIN_CAPACITY * num_tokens / N_EXP)
    cap += cap % 2
    return int(max(cap, MIN_CAPACITY))


def _pass_a_kernel(cap, x_ref, wg_ref, meta_ref, stats_ref, c0_ref, c1_ref):
    i = pl.program_id(0)
    T = x_ref.shape[0]
    E = N_EXP

    @pl.when(i == 0)
    def _init():
        c0_ref[...] = jnp.zeros_like(c0_ref)
        c1_ref[...] = jnp.zeros_like(c1_ref)

    logits = lax.dot_general(
        x_ref[:, 0, :], wg_ref[...],
        dimension_numbers=(((1,), (1,)), ((), ())),
        preferred_element_type=jnp.float32,
    )

    eidx = lax.broadcasted_iota(jnp.int32, (T, E), 1)
    neg_inf = jnp.float32(-jnp.inf)

    m0 = jnp.max(logits, axis=1, keepdims=True)
    e0 = jnp.min(jnp.where(logits == m0, eidx, E), axis=1, keepdims=True)
    l1 = jnp.where(eidx == e0, neg_inf, logits)
    m1 = jnp.max(l1, axis=1, keepdims=True)
    e1 = jnp.min(jnp.where(l1 == m1, eidx, E), axis=1, keepdims=True)

    z = jnp.exp(m1 - m0)
    w0 = 1.0 / (1.0 + z)
    w1 = z / (1.0 + z)

    oh0 = (eidx == e0).astype(jnp.int32)
    oh1 = (eidx == e1).astype(jnp.int32)
    ir = lax.broadcasted_iota(jnp.int32, (T, T), 0)
    ic = lax.broadcasted_iota(jnp.int32, (T, T), 1)
    tril = (ir >= ic).astype(jnp.float32)
    both = jnp.concatenate([oh0, oh1], axis=1).astype(jnp.float32)
    cs = jnp.dot(tril, both, preferred_element_type=jnp.float32)
    cs = cs.astype(jnp.int32)
    cs0 = cs[:, :E]
    cs1 = cs[:, E:]
    carry0 = c0_ref[...]
    carry1 = c1_ref[...]
    r0 = jnp.sum(oh0 * (carry0 + cs0), axis=1, keepdims=True) - 1
    p1 = jnp.sum(oh1 * (carry1 + cs1), axis=1, keepdims=True) - 1
    new_c0 = carry0 + cs0[T - 1:T, :]
    new_c1 = carry1 + cs1[T - 1:T, :]
    c0_ref[...] = new_c0
    c1_ref[...] = new_c1

    zf = jnp.zeros((T, 1), jnp.float32)
    mcols = jnp.concatenate(
        [e0.astype(jnp.float32), e1.astype(jnp.float32),
         r0.astype(jnp.float32), p1.astype(jnp.float32), w0, w1, zf, zf],
        axis=1)
    eye = (ir == ic).astype(jnp.float32)
    meta_ref[...] = lax.dot_general(
        mcols, eye, dimension_numbers=(((0,), (0,)), ((), ())),
        precision=lax.Precision.HIGHEST,
        preferred_element_type=jnp.float32)

    zi = jnp.zeros((1, 16 - E), jnp.int32)
    row0 = jnp.concatenate([new_c0, zi], axis=1)
    row1 = jnp.concatenate(
        [jnp.minimum(new_c0 + new_c1, jnp.int32(cap)), zi], axis=1)
    stats_ref[...] = jnp.concatenate([row0, row1], axis=0)


def _sc_scatter_body(N, cap, meta_hbm, stats_hbm, a_init_hbm, v_init_hbm,
                     a_out, v_out,
                     meta_v, tot_v, si0_v, si1_v, ai0_v, ai1_v,
                     vv0_v, vv1_v, a_sh, v_sh, sem):
    E = N_EXP
    L = SC_LANES
    tok_per_tile = N // SC_TILES
    nch = tok_per_tile // L
    tbl = E * cap

    wid = lax.axis_index("s") * SC_CORES + lax.axis_index("c")
    base = wid * tok_per_tile

    @pl.when(wid == 0)
    def _init_tables():
        pltpu.sync_copy(a_init_hbm, a_sh)
        pltpu.sync_copy(v_init_hbm, v_sh)

    copies = [
        pltpu.make_async_copy(
            meta_hbm.at[r, pl.ds(base, tok_per_tile)],
            meta_v.at[pl.ds(r * tok_per_tile, tok_per_tile)],
            sem)
        for r in range(6)
    ]
    copies.append(
        pltpu.make_async_copy(stats_hbm.at[0, pl.ds(0, 16)], tot_v, sem))
    for cp in copies:
        cp.start()
    for cp in copies:
        cp.wait()

    for c in range(nch):
        off = c * L
        e0 = meta_v[pl.ds(0 * tok_per_tile + off, L)].astype(jnp.int32)
        e1 = meta_v[pl.ds(1 * tok_per_tile + off, L)].astype(jnp.int32)
        r0 = meta_v[pl.ds(2 * tok_per_tile + off, L)].astype(jnp.int32)
        p1 = meta_v[pl.ds(3 * tok_per_tile + off, L)].astype(jnp.int32)
        w0 = meta_v[pl.ds(4 * tok_per_tile + off, L)]
        w1 = meta_v[pl.ds(5 * tok_per_tile + off, L)]

        totv = tot_v[...]
        tot_e1 = lax.gather(
            totv, e1[:, None],
            dimension_numbers=lax.GatherDimensionNumbers(
                offset_dims=(), collapsed_slice_dims=(0,),
                start_index_map=(0,)),
            slice_sizes=(1,),
            mode=lax.GatherScatterMode.PROMISE_IN_BOUNDS)
        rank1 = p1 + tot_e1
        n = base + off + lax.iota(jnp.int32, L)

        keep0 = r0 < cap
        keep1 = rank1 < cap
        dump0 = tbl + wid * (2 * tok_per_tile) + off + lax.iota(jnp.int32, L)
        dump1 = dump0 + tok_per_tile
        slot0 = jnp.where(keep0, e0 * cap + r0, dump0)
        slot1 = jnp.where(keep1, e1 * cap + rank1, dump1)

        si0_v[pl.ds(off, L)] = slot0
        si1_v[pl.ds(off, L)] = slot1
        ai0_v[pl.ds(off, L)] = n
        ai1_v[pl.ds(off, L)] = n
        vv0_v[pl.ds(off, L)] = w0
        vv1_v[pl.ds(off, L)] = w1

    plsc.subcore_barrier()
    scat = [
        pltpu.make_async_copy(ai0_v, a_sh.at[si0_v], sem),
        pltpu.make_async_copy(ai1_v, a_sh.at[si1_v], sem),
        pltpu.make_async_copy(vv0_v, v_sh.at[si0_v], sem),
        pltpu.make_async_copy(vv1_v, v_sh.at[si1_v], sem),
    ]
    for cp in scat:
        cp.start()
    for cp in scat:
        cp.wait()
    plsc.subcore_barrier()

    @pl.when(wid == 0)
    def _ship_out():
        out_copies = [
            pltpu.make_async_copy(a_sh.at[pl.ds(0, tbl)], a_out, sem),
            pltpu.make_async_copy(v_sh.at[pl.ds(0, tbl)], v_out, sem),
        ]
        for cp in out_copies:
            cp.start()
        for cp in out_copies:
            cp.wait()


def _pass_c_kernel(cap, a_ref, v_ref, cb_ref, mask_ref):
    T = cb_ref.shape[0]
    E = N_EXP
    i = pl.program_id(0)

    a2 = a_ref[...]
    v2 = v_ref[...]
    nz2 = v2 != 0.0

    n3 = (i * T
          + lax.broadcasted_iota(jnp.int32, (T, E, cap), 0))
    hit = a2[None, :, :] == n3
    cbv = jnp.where(hit, v2[None, :, :], jnp.float32(0.0))
    cb_ref[...] = cbv
    mask_ref[...] = (hit & nz2[None, :, :]).astype(jnp.int8)


def kernel(x, W_g):
    N = x.shape[0]
    D = x.shape[2]
    E = N_EXP
    cap = _capacity(N)

    TA = 512
    nb_a = N // TA
    meta, stats = pl.pallas_call(
        functools.partial(_pass_a_kernel, cap),
        grid=(nb_a,),
        in_specs=[
            pl.BlockSpec((TA, 1, D), lambda i: (i, 0, 0)),
            pl.BlockSpec((E, D), lambda i: (0, 0)),
        ],
        out_specs=[
            pl.BlockSpec((8, TA), lambda i: (0, i)),
            pl.BlockSpec((2, 16), lambda i: (0, 0)),
        ],
        out_shape=[
            jax.ShapeDtypeStruct((8, N), jnp.float32),
            jax.ShapeDtypeStruct((2, 16), jnp.int32),
        ],
        scratch_shapes=[
            pltpu.VMEM((1, E), jnp.int32),
            pltpu.VMEM((1, E), jnp.int32),
        ],
    )(x, W_g)

    tok_per_tile = N // SC_TILES
    tbl_sz = E * cap + SC_TILES * 2 * tok_per_tile
    mesh = plsc.VectorSubcoreMesh(
        core_axis_name="c", subcore_axis_name="s",
        num_cores=SC_CORES, num_subcores=SC_SUBCORES)
    sc_scatter = pl.kernel(
        functools.partial(_sc_scatter_body, N, cap),
        out_type=(
            jax.ShapeDtypeStruct((E * cap,), jnp.int32),
            jax.ShapeDtypeStruct((E * cap,), jnp.float32),
        ),
        mesh=mesh,
        scratch_types=[
            pltpu.VMEM((6 * tok_per_tile,), jnp.float32),
            pltpu.VMEM((16,), jnp.int32),
            pltpu.VMEM((tok_per_tile,), jnp.int32),
            pltpu.VMEM((tok_per_tile,), jnp.int32),
            pltpu.VMEM((tok_per_tile,), jnp.int32),
            pltpu.VMEM((tok_per_tile,), jnp.int32),
            pltpu.VMEM((tok_per_tile,), jnp.float32),
            pltpu.VMEM((tok_per_tile,), jnp.float32),
            pltpu.VMEM_SHARED((tbl_sz,), jnp.int32),
            pltpu.VMEM_SHARED((tbl_sz,), jnp.float32),
            pltpu.SemaphoreType.DMA,
        ],
    )

    a_init = jnp.full((tbl_sz,), -1, jnp.int32)
    v_init = jnp.zeros((tbl_sz,), jnp.float32)
    a_flat, v_flat = sc_scatter(meta, stats, a_init, v_init)
    a_tbl = a_flat.reshape(E, cap)
    v_tbl = v_flat.reshape(E, cap)

    TC = 64
    nb_c = N // TC
    cb, mask8 = pl.pallas_call(
        functools.partial(_pass_c_kernel, cap),
        grid=(nb_c,),
        in_specs=[
            pl.BlockSpec((E, cap), lambda i: (0, 0)),
            pl.BlockSpec((E, cap), lambda i: (0, 0)),
        ],
        out_specs=[
            pl.BlockSpec((TC, E, cap), lambda i: (i, 0, 0)),
            pl.BlockSpec((TC, E, cap), lambda i: (i, 0, 0)),
        ],
        out_shape=[
            jax.ShapeDtypeStruct((N, E, cap), jnp.float32),
            jax.ShapeDtypeStruct((N, E, cap), jnp.int8),
        ],
    )(a_tbl, v_tbl)

    used_capacity = stats[1, :E]
    return used_capacity, cb, mask8.astype(jnp.bool_)


# speedup vs baseline: 1.0269x; 1.0269x over previous
import functools
import math

import jax
import jax.numpy as jnp
from jax import lax
from jax.experimental import pallas as pl
from jax.experimental.pallas import tpu as pltpu
from jax.experimental.pallas import tpu_sc as plsc

N_EXP = 8
TOP_K = 2
TRAIN_CAPACITY = 1.25
MIN_CAPACITY = 4

SC_CORES = 1
SC_SUBCORES = 16
SC_LANES = 16
SC_TILES = SC_CORES * SC_SUBCORES


def _capacity(num_tokens: int) -> int:
    cap = math.floor(TOP_K * TRAIN_CAPACITY * num_tokens / N_EXP)
    cap += cap % 2
    return int(max(cap, MIN_CAPACITY))


def _pass_a_kernel(cap, x_ref, wg_ref, meta_ref, stats_ref, c0_ref, c1_ref):
    i = pl.program_id(0)
    T = x_ref.shape[0]
    E = N_EXP

    @pl.when(i == 0)
    def _init():
        c0_ref[...] = jnp.zeros_like(c0_ref)
        c1_ref[...] = jnp.zeros_like(c1_ref)

    logits = lax.dot_general(
        x_ref[:, 0, :], wg_ref[...],
        dimension_numbers=(((1,), (1,)), ((), ())),
        preferred_element_type=jnp.float32,
    )

    eidx = lax.broadcasted_iota(jnp.int32, (T, E), 1)
    neg_inf = jnp.float32(-jnp.inf)

    m0 = jnp.max(logits, axis=1, keepdims=True)
    e0 = jnp.min(jnp.where(logits == m0, eidx, E), axis=1, keepdims=True)
    l1 = jnp.where(eidx == e0, neg_inf, logits)
    m1 = jnp.max(l1, axis=1, keepdims=True)
    e1 = jnp.min(jnp.where(l1 == m1, eidx, E), axis=1, keepdims=True)

    z = jnp.exp(m1 - m0)
    w0 = 1.0 / (1.0 + z)
    w1 = z / (1.0 + z)

    oh0 = (eidx == e0).astype(jnp.int32)
    oh1 = (eidx == e1).astype(jnp.int32)
    ir = lax.broadcasted_iota(jnp.int32, (T, T), 0)
    ic = lax.broadcasted_iota(jnp.int32, (T, T), 1)
    tril = (ir >= ic).astype(jnp.float32)
    both = jnp.concatenate([oh0, oh1], axis=1).astype(jnp.float32)
    cs = jnp.dot(tril, both, preferred_element_type=jnp.float32)
    cs = cs.astype(jnp.int32)
    cs0 = cs[:, :E]
    cs1 = cs[:, E:]
    carry0 = c0_ref[...]
    carry1 = c1_ref[...]
    r0 = jnp.sum(oh0 * (carry0 + cs0), axis=1, keepdims=True) - 1
    p1 = jnp.sum(oh1 * (carry1 + cs1), axis=1, keepdims=True) - 1
    new_c0 = carry0 + cs0[T - 1:T, :]
    new_c1 = carry1 + cs1[T - 1:T, :]
    c0_ref[...] = new_c0
    c1_ref[...] = new_c1

    zf = jnp.zeros((T, 1), jnp.float32)
    mcols = jnp.concatenate(
        [e0.astype(jnp.float32), e1.astype(jnp.float32),
         r0.astype(jnp.float32), p1.astype(jnp.float32), w0, w1, zf, zf],
        axis=1)
    eye = (ir == ic).astype(jnp.float32)
    meta_ref[...] = lax.dot_general(
        mcols, eye, dimension_numbers=(((0,), (0,)), ((), ())),
        precision=lax.Precision.HIGHEST,
        preferred_element_type=jnp.float32)

    zi = jnp.zeros((1, 16 - E), jnp.int32)
    row0 = jnp.concatenate([new_c0, zi], axis=1)
    row1 = jnp.concatenate(
        [jnp.minimum(new_c0 + new_c1, jnp.int32(cap)), zi], axis=1)
    stats_ref[...] = jnp.concatenate([row0, row1], axis=0)


def _sc_scatter_body(N, cap, meta_hbm, stats_hbm, a_init_hbm, v_init_hbm,
                     a_out, v_out,
                     meta_v, tot_v, si0_v, si1_v, ai0_v, ai1_v,
                     vv0_v, vv1_v, a_sh, v_sh, sem):
    E = N_EXP
    L = SC_LANES
    tok_per_tile = N // SC_TILES
    nch = tok_per_tile // L
    tbl = E * cap

    wid = lax.axis_index("s") * SC_CORES + lax.axis_index("c")
    base = wid * tok_per_tile

    @pl.when(wid == 0)
    def _init_tables():
        pltpu.sync_copy(a_init_hbm, a_sh)
        pltpu.sync_copy(v_init_hbm, v_sh)

    copies = [
        pltpu.make_async_copy(
            meta_hbm.at[r, pl.ds(base, tok_per_tile)],
            meta_v.at[pl.ds(r * tok_per_tile, tok_per_tile)],
            sem)
        for r in range(6)
    ]
    copies.append(
        pltpu.make_async_copy(stats_hbm.at[0, pl.ds(0, 16)], tot_v, sem))
    for cp in copies:
        cp.start()
    for cp in copies:
        cp.wait()

    for c in range(nch):
        off = c * L
        e0 = meta_v[pl.ds(0 * tok_per_tile + off, L)].astype(jnp.int32)
        e1 = meta_v[pl.ds(1 * tok_per_tile + off, L)].astype(jnp.int32)
        r0 = meta_v[pl.ds(2 * tok_per_tile + off, L)].astype(jnp.int32)
        p1 = meta_v[pl.ds(3 * tok_per_tile + off, L)].astype(jnp.int32)
        w0 = meta_v[pl.ds(4 * tok_per_tile + off, L)]
        w1 = meta_v[pl.ds(5 * tok_per_tile + off, L)]

        totv = tot_v[...]
        tot_e1 = lax.gather(
            totv, e1[:, None],
            dimension_numbers=lax.GatherDimensionNumbers(
                offset_dims=(), collapsed_slice_dims=(0,),
                start_index_map=(0,)),
            slice_sizes=(1,),
            mode=lax.GatherScatterMode.PROMISE_IN_BOUNDS)
        rank1 = p1 + tot_e1
        n = base + off + lax.iota(jnp.int32, L)

        keep0 = r0 < cap
        keep1 = rank1 < cap
        dump0 = tbl + wid * (2 * tok_per_tile) + off + lax.iota(jnp.int32, L)
        dump1 = dump0 + tok_per_tile
        slot0 = jnp.where(keep0, e0 * cap + r0, dump0)
        slot1 = jnp.where(keep1, e1 * cap + rank1, dump1)

        si0_v[pl.ds(off, L)] = slot0
        si1_v[pl.ds(off, L)] = slot1
        ai0_v[pl.ds(off, L)] = n
        ai1_v[pl.ds(off, L)] = n
        vv0_v[pl.ds(off, L)] = w0
        vv1_v[pl.ds(off, L)] = w1

    plsc.subcore_barrier()
    scat = [
        pltpu.make_async_copy(ai0_v, a_sh.at[si0_v], sem),
        pltpu.make_async_copy(ai1_v, a_sh.at[si1_v], sem),
        pltpu.make_async_copy(vv0_v, v_sh.at[si0_v], sem),
        pltpu.make_async_copy(vv1_v, v_sh.at[si1_v], sem),
    ]
    for cp in scat:
        cp.start()
    for cp in scat:
        cp.wait()
    plsc.subcore_barrier()

    @pl.when(wid == 0)
    def _ship_out():
        out_copies = [
            pltpu.make_async_copy(a_sh.at[pl.ds(0, tbl)], a_out, sem),
            pltpu.make_async_copy(v_sh.at[pl.ds(0, tbl)], v_out, sem),
        ]
        for cp in out_copies:
            cp.start()
        for cp in out_copies:
            cp.wait()


def _pass_c_kernel(cap, a_ref, v_ref, cb_ref, mask_ref):
    T = cb_ref.shape[0]
    E = N_EXP
    i = pl.program_id(0)

    a2 = a_ref[...]
    v2 = v_ref[...]
    nz2 = v2 != 0.0

    n3 = (i * T
          + lax.broadcasted_iota(jnp.int32, (T, E, cap), 0))
    hit = a2[None, :, :] == n3
    cbv = jnp.where(hit, v2[None, :, :], jnp.float32(0.0))
    cb_ref[...] = cbv
    mask_ref[...] = (hit & nz2[None, :, :]).astype(jnp.int8)


def kernel(x, W_g):
    N = x.shape[0]
    D = x.shape[2]
    E = N_EXP
    cap = _capacity(N)

    TA = 1024
    nb_a = N // TA
    meta, stats = pl.pallas_call(
        functools.partial(_pass_a_kernel, cap),
        grid=(nb_a,),
        in_specs=[
            pl.BlockSpec((TA, 1, D), lambda i: (i, 0, 0)),
            pl.BlockSpec((E, D), lambda i: (0, 0)),
        ],
        out_specs=[
            pl.BlockSpec((8, TA), lambda i: (0, i)),
            pl.BlockSpec((2, 16), lambda i: (0, 0)),
        ],
        out_shape=[
            jax.ShapeDtypeStruct((8, N), jnp.float32),
            jax.ShapeDtypeStruct((2, 16), jnp.int32),
        ],
        scratch_shapes=[
            pltpu.VMEM((1, E), jnp.int32),
            pltpu.VMEM((1, E), jnp.int32),
        ],
    )(x, W_g)

    tok_per_tile = N // SC_TILES
    tbl_sz = E * cap + SC_TILES * 2 * tok_per_tile
    mesh = plsc.VectorSubcoreMesh(
        core_axis_name="c", subcore_axis_name="s",
        num_cores=SC_CORES, num_subcores=SC_SUBCORES)
    sc_scatter = pl.kernel(
        functools.partial(_sc_scatter_body, N, cap),
        out_type=(
            jax.ShapeDtypeStruct((E * cap,), jnp.int32),
            jax.ShapeDtypeStruct((E * cap,), jnp.float32),
        ),
        mesh=mesh,
        scratch_types=[
            pltpu.VMEM((6 * tok_per_tile,), jnp.float32),
            pltpu.VMEM((16,), jnp.int32),
            pltpu.VMEM((tok_per_tile,), jnp.int32),
            pltpu.VMEM((tok_per_tile,), jnp.int32),
            pltpu.VMEM((tok_per_tile,), jnp.int32),
            pltpu.VMEM((tok_per_tile,), jnp.int32),
            pltpu.VMEM((tok_per_tile,), jnp.float32),
            pltpu.VMEM((tok_per_tile,), jnp.float32),
            pltpu.VMEM_SHARED((tbl_sz,), jnp.int32),
            pltpu.VMEM_SHARED((tbl_sz,), jnp.float32),
            pltpu.SemaphoreType.DMA,
        ],
    )

    a_init = jnp.full((tbl_sz,), -1, jnp.int32)
    v_init = jnp.zeros((tbl_sz,), jnp.float32)
    a_flat, v_flat = sc_scatter(meta, stats, a_init, v_init)
    a_tbl = a_flat.reshape(E, cap)
    v_tbl = v_flat.reshape(E, cap)

    TC = 128
    nb_c = N // TC
    cb, mask8 = pl.pallas_call(
        functools.partial(_pass_c_kernel, cap),
        grid=(nb_c,),
        in_specs=[
            pl.BlockSpec((E, cap), lambda i: (0, 0)),
            pl.BlockSpec((E, cap), lambda i: (0, 0)),
        ],
        out_specs=[
            pl.BlockSpec((TC, E, cap), lambda i: (i, 0, 0)),
            pl.BlockSpec((TC, E, cap), lambda i: (i, 0, 0)),
        ],
        out_shape=[
            jax.ShapeDtypeStruct((N, E, cap), jnp.float32),
            jax.ShapeDtypeStruct((N, E, cap), jnp.int8),
        ],
    )(a_tbl, v_tbl)

    used_capacity = stats[1, :E]
    return used_capacity, cb, mask8.astype(jnp.bool_)


# speedup vs baseline: 1.0554x; 1.0277x over previous
import functools
import math

import jax
import jax.numpy as jnp
from jax import lax
from jax.experimental import pallas as pl
from jax.experimental.pallas import tpu as pltpu
from jax.experimental.pallas import tpu_sc as plsc

N_EXP = 8
TOP_K = 2
TRAIN_CAPACITY = 1.25
MIN_CAPACITY = 4

SC_CORES = 1
SC_SUBCORES = 16
SC_LANES = 16
SC_TILES = SC_CORES * SC_SUBCORES


def _capacity(num_tokens: int) -> int:
    cap = math.floor(TOP_K * TRAIN_CAPACITY * num_tokens / N_EXP)
    cap += cap % 2
    return int(max(cap, MIN_CAPACITY))


def _pass_a_kernel(cap, x_ref, wg_ref, meta_ref, stats_ref, c0_ref, c1_ref):
    i = pl.program_id(0)
    T = x_ref.shape[0]
    E = N_EXP

    @pl.when(i == 0)
    def _init():
        c0_ref[...] = jnp.zeros_like(c0_ref)
        c1_ref[...] = jnp.zeros_like(c1_ref)

    logits = lax.dot_general(
        x_ref[:, 0, :], wg_ref[...],
        dimension_numbers=(((1,), (1,)), ((), ())),
        preferred_element_type=jnp.float32,
    )

    eidx = lax.broadcasted_iota(jnp.int32, (T, E), 1)
    neg_inf = jnp.float32(-jnp.inf)

    m0 = jnp.max(logits, axis=1, keepdims=True)
    e0 = jnp.min(jnp.where(logits == m0, eidx, E), axis=1, keepdims=True)
    l1 = jnp.where(eidx == e0, neg_inf, logits)
    m1 = jnp.max(l1, axis=1, keepdims=True)
    e1 = jnp.min(jnp.where(l1 == m1, eidx, E), axis=1, keepdims=True)

    z = jnp.exp(m1 - m0)
    w0 = 1.0 / (1.0 + z)
    w1 = z / (1.0 + z)

    oh0 = (eidx == e0).astype(jnp.int32)
    oh1 = (eidx == e1).astype(jnp.int32)
    ir = lax.broadcasted_iota(jnp.int32, (T, T), 0)
    ic = lax.broadcasted_iota(jnp.int32, (T, T), 1)
    tril = (ir >= ic).astype(jnp.float32)
    both = jnp.concatenate([oh0, oh1], axis=1).astype(jnp.float32)
    cs = jnp.dot(tril, both, preferred_element_type=jnp.float32)
    cs = cs.astype(jnp.int32)
    cs0 = cs[:, :E]
    cs1 = cs[:, E:]
    carry0 = c0_ref[...]
    carry1 = c1_ref[...]
    r0 = jnp.sum(oh0 * (carry0 + cs0), axis=1, keepdims=True) - 1
    p1 = jnp.sum(oh1 * (carry1 + cs1), axis=1, keepdims=True) - 1
    new_c0 = carry0 + cs0[T - 1:T, :]
    new_c1 = carry1 + cs1[T - 1:T, :]
    c0_ref[...] = new_c0
    c1_ref[...] = new_c1

    zf = jnp.zeros((T, 1), jnp.float32)
    mcols = jnp.concatenate(
        [e0.astype(jnp.float32), e1.astype(jnp.float32),
         r0.astype(jnp.float32), p1.astype(jnp.float32), w0, w1, zf, zf],
        axis=1)
    eye = (ir == ic).astype(jnp.float32)
    meta_ref[...] = lax.dot_general(
        mcols, eye, dimension_numbers=(((0,), (0,)), ((), ())),
        precision=lax.Precision.HIGHEST,
        preferred_element_type=jnp.float32)

    zi = jnp.zeros((1, 16 - E), jnp.int32)
    row0 = jnp.concatenate([new_c0, zi], axis=1)
    row1 = jnp.concatenate(
        [jnp.minimum(new_c0 + new_c1, jnp.int32(cap)), zi], axis=1)
    stats_ref[...] = jnp.concatenate([row0, row1], axis=0)


def _sc_scatter_body(N, cap, meta_hbm, stats_hbm, a_init_hbm, v_init_hbm,
                     a_out, v_out,
                     meta_v, tot_v, si0_v, si1_v, ai0_v, ai1_v,
                     vv0_v, vv1_v, a_sh, v_sh, sem):
    E = N_EXP
    L = SC_LANES
    tok_per_tile = N // SC_TILES
    nch = tok_per_tile // L
    tbl = E * cap

    wid = lax.axis_index("s") * SC_CORES + lax.axis_index("c")
    base = wid * tok_per_tile

    @pl.when(wid == 0)
    def _init_tables():
        pltpu.sync_copy(a_init_hbm, a_sh)
        pltpu.sync_copy(v_init_hbm, v_sh)

    copies = [
        pltpu.make_async_copy(
            meta_hbm.at[r, pl.ds(base, tok_per_tile)],
            meta_v.at[pl.ds(r * tok_per_tile, tok_per_tile)],
            sem)
        for r in range(6)
    ]
    copies.append(
        pltpu.make_async_copy(stats_hbm.at[0, pl.ds(0, 16)], tot_v, sem))
    for cp in copies:
        cp.start()
    for cp in copies:
        cp.wait()

    for c in range(nch):
        off = c * L
        e0 = meta_v[pl.ds(0 * tok_per_tile + off, L)].astype(jnp.int32)
        e1 = meta_v[pl.ds(1 * tok_per_tile + off, L)].astype(jnp.int32)
        r0 = meta_v[pl.ds(2 * tok_per_tile + off, L)].astype(jnp.int32)
        p1 = meta_v[pl.ds(3 * tok_per_tile + off, L)].astype(jnp.int32)
        w0 = meta_v[pl.ds(4 * tok_per_tile + off, L)]
        w1 = meta_v[pl.ds(5 * tok_per_tile + off, L)]

        totv = tot_v[...]
        tot_e1 = lax.gather(
            totv, e1[:, None],
            dimension_numbers=lax.GatherDimensionNumbers(
                offset_dims=(), collapsed_slice_dims=(0,),
                start_index_map=(0,)),
            slice_sizes=(1,),
            mode=lax.GatherScatterMode.PROMISE_IN_BOUNDS)
        rank1 = p1 + tot_e1
        n = base + off + lax.iota(jnp.int32, L)

        keep0 = r0 < cap
        keep1 = rank1 < cap
        dump0 = tbl + wid * (2 * tok_per_tile) + off + lax.iota(jnp.int32, L)
        dump1 = dump0 + tok_per_tile
        slot0 = jnp.where(keep0, e0 * cap + r0, dump0)
        slot1 = jnp.where(keep1, e1 * cap + rank1, dump1)

        si0_v[pl.ds(off, L)] = slot0
        si1_v[pl.ds(off, L)] = slot1
        ai0_v[pl.ds(off, L)] = n
        ai1_v[pl.ds(off, L)] = n
        vv0_v[pl.ds(off, L)] = w0
        vv1_v[pl.ds(off, L)] = w1

    plsc.subcore_barrier()
    scat = [
        pltpu.make_async_copy(ai0_v, a_sh.at[si0_v], sem),
        pltpu.make_async_copy(ai1_v, a_sh.at[si1_v], sem),
        pltpu.make_async_copy(vv0_v, v_sh.at[si0_v], sem),
        pltpu.make_async_copy(vv1_v, v_sh.at[si1_v], sem),
    ]
    for cp in scat:
        cp.start()
    for cp in scat:
        cp.wait()
    plsc.subcore_barrier()

    @pl.when(wid == 0)
    def _ship_out():
        out_copies = [
            pltpu.make_async_copy(a_sh.at[pl.ds(0, tbl)], a_out, sem),
            pltpu.make_async_copy(v_sh.at[pl.ds(0, tbl)], v_out, sem),
        ]
        for cp in out_copies:
            cp.start()
        for cp in out_copies:
            cp.wait()


def _pass_c_kernel(cap, a_ref, v_ref, cb_ref, mask_ref):
    T = cb_ref.shape[0]
    E = N_EXP
    i = pl.program_id(0)

    a2 = a_ref[...]
    v2 = v_ref[...]
    nz2 = v2 != 0.0

    n3 = (i * T
          + lax.broadcasted_iota(jnp.int32, (T, E, cap), 0))
    hit = a2[None, :, :] == n3
    cbv = jnp.where(hit, v2[None, :, :], jnp.float32(0.0))
    cb_ref[...] = cbv
    mask_ref[...] = (hit & nz2[None, :, :]).astype(jnp.int8)


def kernel(x, W_g):
    N = x.shape[0]
    D = x.shape[2]
    E = N_EXP
    cap = _capacity(N)

    TA = 512
    nb_a = N // TA
    meta, stats = pl.pallas_call(
        functools.partial(_pass_a_kernel, cap),
        grid=(nb_a,),
        in_specs=[
            pl.BlockSpec((TA, 1, D), lambda i: (i, 0, 0)),
            pl.BlockSpec((E, D), lambda i: (0, 0)),
        ],
        out_specs=[
            pl.BlockSpec((8, TA), lambda i: (0, i)),
            pl.BlockSpec((2, 16), lambda i: (0, 0)),
        ],
        out_shape=[
            jax.ShapeDtypeStruct((8, N), jnp.float32),
            jax.ShapeDtypeStruct((2, 16), jnp.int32),
        ],
        scratch_shapes=[
            pltpu.VMEM((1, E), jnp.int32),
            pltpu.VMEM((1, E), jnp.int32),
        ],
    )(x, W_g)

    tok_per_tile = N // SC_TILES
    tbl_sz = E * cap + SC_TILES * 2 * tok_per_tile
    mesh = plsc.VectorSubcoreMesh(
        core_axis_name="c", subcore_axis_name="s",
        num_cores=SC_CORES, num_subcores=SC_SUBCORES)
    sc_scatter = pl.kernel(
        functools.partial(_sc_scatter_body, N, cap),
        out_type=(
            jax.ShapeDtypeStruct((E * cap,), jnp.int32),
            jax.ShapeDtypeStruct((E * cap,), jnp.float32),
        ),
        mesh=mesh,
        scratch_types=[
            pltpu.VMEM((6 * tok_per_tile,), jnp.float32),
            pltpu.VMEM((16,), jnp.int32),
            pltpu.VMEM((tok_per_tile,), jnp.int32),
            pltpu.VMEM((tok_per_tile,), jnp.int32),
            pltpu.VMEM((tok_per_tile,), jnp.int32),
            pltpu.VMEM((tok_per_tile,), jnp.int32),
            pltpu.VMEM((tok_per_tile,), jnp.float32),
            pltpu.VMEM((tok_per_tile,), jnp.float32),
            pltpu.VMEM_SHARED((tbl_sz,), jnp.int32),
            pltpu.VMEM_SHARED((tbl_sz,), jnp.float32),
            pltpu.SemaphoreType.DMA,
        ],
    )

    a_init = jnp.full((tbl_sz,), -1, jnp.int32)
    v_init = jnp.zeros((tbl_sz,), jnp.float32)
    a_flat, v_flat = sc_scatter(meta, stats, a_init, v_init)
    a_tbl = a_flat.reshape(E, cap)
    v_tbl = v_flat.reshape(E, cap)

    TC = 128
    nb_c = N // TC
    cb, mask8 = pl.pallas_call(
        functools.partial(_pass_c_kernel, cap),
        grid=(nb_c,),
        in_specs=[
            pl.BlockSpec((E, cap), lambda i: (0, 0)),
            pl.BlockSpec((E, cap), lambda i: (0, 0)),
        ],
        out_specs=[
            pl.BlockSpec((TC, E, cap), lambda i: (i, 0, 0)),
            pl.BlockSpec((TC, E, cap), lambda i: (i, 0, 0)),
        ],
        out_shape=[
            jax.ShapeDtypeStruct((N, E, cap), jnp.float32),
            jax.ShapeDtypeStruct((N, E, cap), jnp.int8),
        ],
    )(a_tbl, v_tbl)

    used_capacity = stats[1, :E]
    return used_capacity, cb, mask8.astype(jnp.bool_)


# speedup vs baseline: 1.0855x; 1.0285x over previous
import functools
import math

import jax
import jax.numpy as jnp
import numpy as np
from jax import lax
from jax.experimental import pallas as pl
from jax.experimental.pallas import tpu as pltpu
from jax.experimental.pallas import tpu_sc as plsc

N_EXP = 8
TOP_K = 2
TRAIN_CAPACITY = 1.25
MIN_CAPACITY = 4

SC_CORES = 1
SC_SUBCORES = 16
SC_LANES = 16
SC_TILES = SC_CORES * SC_SUBCORES


def _capacity(num_tokens: int) -> int:
    cap = math.floor(TOP_K * TRAIN_CAPACITY * num_tokens / N_EXP)
    cap += cap % 2
    return int(max(cap, MIN_CAPACITY))


def _pass_a_kernel(cap, x_ref, wg_ref, meta_ref, stats_ref, c0_ref, c1_ref,
                   tril_ref, eye_ref):
    i = pl.program_id(0)
    T = x_ref.shape[0]
    E = N_EXP

    @pl.when(i == 0)
    def _init():
        c0_ref[...] = jnp.zeros_like(c0_ref)
        c1_ref[...] = jnp.zeros_like(c1_ref)
        ir0 = lax.broadcasted_iota(jnp.int32, (T, T), 0)
        ic0 = lax.broadcasted_iota(jnp.int32, (T, T), 1)
        tril_ref[...] = (ir0 >= ic0).astype(jnp.float32)
        eye_ref[...] = (ir0 == ic0).astype(jnp.float32)

    logits = lax.dot_general(
        x_ref[:, 0, :], wg_ref[...],
        dimension_numbers=(((1,), (1,)), ((), ())),
        preferred_element_type=jnp.float32,
    )

    eidx = lax.broadcasted_iota(jnp.int32, (T, E), 1)
    neg_inf = jnp.float32(-jnp.inf)

    m0 = jnp.max(logits, axis=1, keepdims=True)
    e0 = jnp.min(jnp.where(logits == m0, eidx, E), axis=1, keepdims=True)
    l1 = jnp.where(eidx == e0, neg_inf, logits)
    m1 = jnp.max(l1, axis=1, keepdims=True)
    e1 = jnp.min(jnp.where(l1 == m1, eidx, E), axis=1, keepdims=True)

    z = jnp.exp(m1 - m0)
    w0 = 1.0 / (1.0 + z)
    w1 = z / (1.0 + z)

    oh0 = (eidx == e0).astype(jnp.int32)
    oh1 = (eidx == e1).astype(jnp.int32)
    both = jnp.concatenate([oh0, oh1], axis=1).astype(jnp.float32)
    cs = jnp.dot(tril_ref[...], both, preferred_element_type=jnp.float32)
    cs = cs.astype(jnp.int32)
    cs0 = cs[:, :E]
    cs1 = cs[:, E:]
    carry0 = c0_ref[...]
    carry1 = c1_ref[...]
    masked = jnp.concatenate(
        [oh0 * (carry0 + cs0), oh1 * (carry1 + cs1)],
        axis=1).astype(jnp.float32)
    sel_r = lax.broadcasted_iota(jnp.int32, (2 * E, 2), 0)
    sel_c = lax.broadcasted_iota(jnp.int32, (2 * E, 2), 1)
    sel = ((sel_r < E) == (sel_c == 0)).astype(jnp.float32)
    ranks = lax.dot_general(
        masked, sel, dimension_numbers=(((1,), (0,)), ((), ())),
        precision=lax.Precision.HIGHEST,
        preferred_element_type=jnp.float32)
    r0 = ranks[:, 0:1].astype(jnp.int32) - 1
    p1 = ranks[:, 1:2].astype(jnp.int32) - 1
    new_c0 = carry0 + cs0[T - 1:T, :]
    new_c1 = carry1 + cs1[T - 1:T, :]
    c0_ref[...] = new_c0
    c1_ref[...] = new_c1

    zf = jnp.zeros((T, 1), jnp.float32)
    mcols = jnp.concatenate(
        [e0.astype(jnp.float32), e1.astype(jnp.float32),
         r0.astype(jnp.float32), p1.astype(jnp.float32), w0, w1, zf, zf],
        axis=1)
    meta_ref[...] = lax.dot_general(
        mcols, eye_ref[...], dimension_numbers=(((0,), (0,)), ((), ())),
        precision=lax.Precision.HIGHEST,
        preferred_element_type=jnp.float32)

    zi = jnp.zeros((1, 16 - E), jnp.int32)
    row0 = jnp.concatenate([new_c0, zi], axis=1)
    row1 = jnp.concatenate(
        [jnp.minimum(new_c0 + new_c1, jnp.int32(cap)), zi], axis=1)
    stats_ref[...] = jnp.concatenate([row0, row1], axis=0)


def _sc_scatter_body(N, cap, meta_hbm, stats_hbm, a_init_hbm, v_init_hbm,
                     a_out, v_out,
                     meta_v, tot_v, si0_v, si1_v, ai0_v, ai1_v,
                     vv0_v, vv1_v, a_sh, v_sh, sem):
    E = N_EXP
    L = SC_LANES
    tok_per_tile = N // SC_TILES
    nch = tok_per_tile // L
    tbl = E * cap

    wid = lax.axis_index("s") * SC_CORES + lax.axis_index("c")
    base = wid * tok_per_tile

    @pl.when(wid == 0)
    def _init_tables():
        pltpu.sync_copy(a_init_hbm, a_sh)
        pltpu.sync_copy(v_init_hbm, v_sh)

    copies = [
        pltpu.make_async_copy(
            meta_hbm.at[r, pl.ds(base, tok_per_tile)],
            meta_v.at[pl.ds(r * tok_per_tile, tok_per_tile)],
            sem)
        for r in range(6)
    ]
    copies.append(
        pltpu.make_async_copy(stats_hbm.at[0, pl.ds(0, 16)], tot_v, sem))
    for cp in copies:
        cp.start()
    for cp in copies:
        cp.wait()

    for c in range(nch):
        off = c * L
        e0 = meta_v[pl.ds(0 * tok_per_tile + off, L)].astype(jnp.int32)
        e1 = meta_v[pl.ds(1 * tok_per_tile + off, L)].astype(jnp.int32)
        r0 = meta_v[pl.ds(2 * tok_per_tile + off, L)].astype(jnp.int32)
        p1 = meta_v[pl.ds(3 * tok_per_tile + off, L)].astype(jnp.int32)
        w0 = meta_v[pl.ds(4 * tok_per_tile + off, L)]
        w1 = meta_v[pl.ds(5 * tok_per_tile + off, L)]

        totv = tot_v[...]
        tot_e1 = lax.gather(
            totv, e1[:, None],
            dimension_numbers=lax.GatherDimensionNumbers(
                offset_dims=(), collapsed_slice_dims=(0,),
                start_index_map=(0,)),
            slice_sizes=(1,),
            mode=lax.GatherScatterMode.PROMISE_IN_BOUNDS)
        rank1 = p1 + tot_e1
        n = base + off + lax.iota(jnp.int32, L)

        keep0 = r0 < cap
        keep1 = rank1 < cap
        dump0 = tbl + wid * (2 * tok_per_tile) + off + lax.iota(jnp.int32, L)
        dump1 = dump0 + tok_per_tile
        slot0 = jnp.where(keep0, e0 * cap + r0, dump0)
        slot1 = jnp.where(keep1, e1 * cap + rank1, dump1)

        si0_v[pl.ds(off, L)] = slot0
        si1_v[pl.ds(off, L)] = slot1
        ai0_v[pl.ds(off, L)] = n
        ai1_v[pl.ds(off, L)] = n
        vv0_v[pl.ds(off, L)] = w0
        vv1_v[pl.ds(off, L)] = w1

    plsc.subcore_barrier()
    scat = [
        pltpu.make_async_copy(ai0_v, a_sh.at[si0_v], sem),
        pltpu.make_async_copy(ai1_v, a_sh.at[si1_v], sem),
        pltpu.make_async_copy(vv0_v, v_sh.at[si0_v], sem),
        pltpu.make_async_copy(vv1_v, v_sh.at[si1_v], sem),
    ]
    for cp in scat:
        cp.start()
    for cp in scat:
        cp.wait()
    plsc.subcore_barrier()

    @pl.when(wid == 0)
    def _ship_out():
        out_copies = [
            pltpu.make_async_copy(a_sh.at[pl.ds(0, tbl)], a_out, sem),
            pltpu.make_async_copy(v_sh.at[pl.ds(0, tbl)], v_out, sem),
        ]
        for cp in out_copies:
            cp.start()
        for cp in out_copies:
            cp.wait()


def _pass_c_kernel(cap, a_ref, v_ref, cb_ref, mask_ref):
    T = cb_ref.shape[0]
    E = N_EXP
    i = pl.program_id(0)

    a2 = a_ref[...]
    v2 = v_ref[...]
    nz2 = v2 != 0.0

    n3 = (i * T
          + lax.broadcasted_iota(jnp.int32, (T, E, cap), 0))
    hit = a2[None, :, :] == n3
    cbv = jnp.where(hit, v2[None, :, :], jnp.float32(0.0))
    cb_ref[...] = cbv
    mask_ref[...] = (hit & nz2[None, :, :]).astype(jnp.int8)


def kernel(x, W_g):
    N = x.shape[0]
    D = x.shape[2]
    E = N_EXP
    cap = _capacity(N)

    TA = 512
    nb_a = N // TA
    meta, stats = pl.pallas_call(
        functools.partial(_pass_a_kernel, cap),
        grid=(nb_a,),
        in_specs=[
            pl.BlockSpec((TA, 1, D), lambda i: (i, 0, 0)),
            pl.BlockSpec((E, D), lambda i: (0, 0)),
        ],
        out_specs=[
            pl.BlockSpec((8, TA), lambda i: (0, i)),
            pl.BlockSpec((2, 16), lambda i: (0, 0)),
        ],
        out_shape=[
            jax.ShapeDtypeStruct((8, N), jnp.float32),
            jax.ShapeDtypeStruct((2, 16), jnp.int32),
        ],
        scratch_shapes=[
            pltpu.VMEM((1, E), jnp.int32),
            pltpu.VMEM((1, E), jnp.int32),
            pltpu.VMEM((TA, TA), jnp.float32),
            pltpu.VMEM((TA, TA), jnp.float32),
        ],
    )(x, W_g)

    tok_per_tile = N // SC_TILES
    tbl_sz = E * cap + SC_TILES * 2 * tok_per_tile
    mesh = plsc.VectorSubcoreMesh(
        core_axis_name="c", subcore_axis_name="s",
        num_cores=SC_CORES, num_subcores=SC_SUBCORES)
    sc_scatter = pl.kernel(
        functools.partial(_sc_scatter_body, N, cap),
        out_type=(
            jax.ShapeDtypeStruct((E * cap,), jnp.int32),
            jax.ShapeDtypeStruct((E * cap,), jnp.float32),
        ),
        mesh=mesh,
        scratch_types=[
            pltpu.VMEM((6 * tok_per_tile,), jnp.float32),
            pltpu.VMEM((16,), jnp.int32),
            pltpu.VMEM((tok_per_tile,), jnp.int32),
            pltpu.VMEM((tok_per_tile,), jnp.int32),
            pltpu.VMEM((tok_per_tile,), jnp.int32),
            pltpu.VMEM((tok_per_tile,), jnp.int32),
            pltpu.VMEM((tok_per_tile,), jnp.float32),
            pltpu.VMEM((tok_per_tile,), jnp.float32),
            pltpu.VMEM_SHARED((tbl_sz,), jnp.int32),
            pltpu.VMEM_SHARED((tbl_sz,), jnp.float32),
            pltpu.SemaphoreType.DMA,
        ],
    )

    a_init = jnp.asarray(np.full((tbl_sz,), -1, np.int32))
    v_init = jnp.asarray(np.zeros((tbl_sz,), np.float32))
    a_flat, v_flat = sc_scatter(meta, stats, a_init, v_init)
    a_tbl = a_flat.reshape(E, cap)
    v_tbl = v_flat.reshape(E, cap)

    TC = 128
    nb_c = N // TC
    cb, mask8 = pl.pallas_call(
        functools.partial(_pass_c_kernel, cap),
        grid=(nb_c,),
        in_specs=[
            pl.BlockSpec((E, cap), lambda i: (0, 0)),
            pl.BlockSpec((E, cap), lambda i: (0, 0)),
        ],
        out_specs=[
            pl.BlockSpec((TC, E, cap), lambda i: (i, 0, 0)),
            pl.BlockSpec((TC, E, cap), lambda i: (i, 0, 0)),
        ],
        out_shape=[
            jax.ShapeDtypeStruct((N, E, cap), jnp.float32),
            jax.ShapeDtypeStruct((N, E, cap), jnp.int8),
        ],
    )(a_tbl, v_tbl)

    used_capacity = stats[1, :E]
    return used_capacity, cb, mask8.astype(jnp.bool_)
